# Initial kernel scaffold; baseline (speedup 1.0000x reference)
import jax
import jax.numpy as jnp
from jax.experimental import pallas as pl


def kernel(node_attr, edge_index, edge_attr, params):
    raise NotImplementedError("write your pallas kernel here")


# speedup vs baseline: 2.1712x; 2.1712x over previous
import functools

import jax
import jax.numpy as jnp
from jax import lax
from jax.experimental import pallas as pl
from jax.experimental.pallas import tpu as pltpu
from jax.experimental.pallas import tpu_sc as plsc

N = 10000
E = 320000
NODE_DIM = 128
EDGE_DIM = 16
EMBED = 128
EPS = 1e-5

NC = 2
NS = 16
NW = NC * NS
CHUNK = 128
E_PAD = 327680
EDGES_PER_TILE = E_PAD // NW
CHUNKS_PER_TILE = EDGES_PER_TILE // CHUNK
N_PAD = 10240
ROWS_PER_TILE = N_PAD // NS
EW = 32


def _sc_mesh():
    return plsc.VectorSubcoreMesh(core_axis_name="c", subcore_axis_name="s")


def _edge_sum_body(ea_hbm, row_hbm, zeros_hbm, out_hbm, row_v, rows_v, acc_sh, sem):
    ci = lax.axis_index("c")
    si = lax.axis_index("s")
    wid = ci * NS + si
    r0 = si * ROWS_PER_TILE
    pltpu.sync_copy(zeros_hbm.at[pl.ds(r0, ROWS_PER_TILE)],
                    acc_sh.at[pl.ds(r0, ROWS_PER_TILE)])
    plsc.subcore_barrier()
    base = wid * EDGES_PER_TILE

    def body(i, carry):
        off = base + i * CHUNK
        pltpu.sync_copy(row_hbm.at[pl.ds(off, CHUNK)], row_v)
        pltpu.async_copy(ea_hbm.at[pl.ds(off, CHUNK)], rows_v, sem).wait()
        pltpu.sync_copy(rows_v, acc_sh.at[row_v], add=True)
        return carry

    lax.fori_loop(0, CHUNKS_PER_TILE, body, 0)
    plsc.subcore_barrier()
    pltpu.sync_copy(acc_sh.at[pl.ds(r0, ROWS_PER_TILE)],
                    out_hbm.at[ci, pl.ds(r0, ROWS_PER_TILE)])


_edge_sum = pl.kernel(
    _edge_sum_body,
    out_type=jax.ShapeDtypeStruct((NC, N_PAD, EW), jnp.float32),
    mesh=_sc_mesh(),
    scratch_types=[
        pltpu.VMEM((CHUNK,), jnp.int32),
        pltpu.VMEM((CHUNK, EW), jnp.float32),
        pltpu.VMEM_SHARED((N_PAD, EW), jnp.float32),
        pltpu.SemaphoreType.DMA,
    ],
    compiler_params=pltpu.CompilerParams(use_tc_tiling_on_sc=False),
)


def _spmm_body(h_hbm, col_hbm, row_hbm, zeros_hbm, out_hbm,
               col_v, row_v, rows_v, acc_sh, sem):
    ci = lax.axis_index("c")
    si = lax.axis_index("s")
    wid = ci * NS + si
    r0 = si * ROWS_PER_TILE
    pltpu.sync_copy(zeros_hbm.at[pl.ds(r0, ROWS_PER_TILE)],
                    acc_sh.at[pl.ds(r0, ROWS_PER_TILE)])
    plsc.subcore_barrier()
    base = wid * EDGES_PER_TILE

    def body(i, carry):
        off = base + i * CHUNK
        pltpu.sync_copy(col_hbm.at[pl.ds(off, CHUNK)], col_v)
        pltpu.sync_copy(row_hbm.at[pl.ds(off, CHUNK)], row_v)
        pltpu.async_copy(h_hbm.at[col_v], rows_v, sem).wait()
        pltpu.sync_copy(rows_v, acc_sh.at[row_v], add=True)
        return carry

    lax.fori_loop(0, CHUNKS_PER_TILE, body, 0)
    plsc.subcore_barrier()
    pltpu.sync_copy(acc_sh.at[pl.ds(r0, ROWS_PER_TILE)],
                    out_hbm.at[ci, pl.ds(r0, ROWS_PER_TILE)])


_spmm = pl.kernel(
    _spmm_body,
    out_type=jax.ShapeDtypeStruct((NC, N_PAD, EMBED), jnp.float32),
    mesh=_sc_mesh(),
    scratch_types=[
        pltpu.VMEM((CHUNK,), jnp.int32),
        pltpu.VMEM((CHUNK,), jnp.int32),
        pltpu.VMEM((CHUNK, EMBED), jnp.float32),
        pltpu.VMEM_SHARED((N_PAD, EMBED), jnp.float32),
        pltpu.SemaphoreType.DMA,
    ],
)


def _bn(y, g, b):
    m = jnp.mean(y, axis=0, keepdims=True)
    v = jnp.mean((y - m) ** 2, axis=0, keepdims=True)
    return g * (y - m) / jnp.sqrt(v + EPS) + b


def _mm(x, w):
    return jnp.dot(x, w, preferred_element_type=jnp.float32)


def _dense1_body(node_ref, w0, b0, g0, beta0, wn, bn_, gnode, bnode, wh, bh,
                 h1_out, xn1_out):
    x = jax.nn.relu(_bn(_mm(node_ref[...], w0[...]) + b0[...], g0[...], beta0[...]))
    h1_out[...] = _mm(x, wh[...]) + bh[...]
    xn1_out[...] = _bn(_mm(x, wn[...]) + bn_[...], gnode[...], bnode[...])


def _mid_a_body(xn_ref, part_ref, spart_ref,
                we, be, gedge, bedge, gnb, bnb, w1, b1, g1, bb1,
                we_n, be_n, gedge_n, bedge_n,
                z2_out, e2_out):
    s = part_ref[0, :N, :] + part_ref[1, :N, :]
    aggr = _bn(s, gnb[...], bnb[...])
    s32 = spart_ref[0, :N, :] + spart_ref[1, :N, :]
    se = s32[:, :EDGE_DIM]
    cnt = s32[:, EDGE_DIM:EDGE_DIM + 1]
    e = _bn(_mm(se, we[...]) + cnt * be[...], gedge[...], bedge[...])
    z = jax.nn.relu(xn_ref[...] + aggr + e)
    z2_out[...] = jax.nn.relu(_bn(_mm(z, w1[...]) + b1[...], g1[...], bb1[...]))
    e2_out[...] = _bn(_mm(se, we_n[...]) + cnt * be_n[...], gedge_n[...], bedge_n[...])


def _mid_b_body(z2_ref, w2, b2, g2, bb2, wn, bn_, gnode, bnode, wh, bh,
                h2_out, xn2_out):
    x = jax.nn.relu(_bn(_mm(z2_ref[...], w2[...]) + b2[...], g2[...], bb2[...]))
    h2_out[...] = _mm(x, wh[...]) + bh[...]
    xn2_out[...] = _bn(_mm(x, wn[...]) + bn_[...], gnode[...], bnode[...])


def _dense_final_body(xn_ref, e_ref, part_ref,
                      gnb, bnb, w1, b1, g1, bb1, w2, b2, g2, bb2,
                      x_out):
    s = part_ref[0, :N, :] + part_ref[1, :N, :]
    aggr = _bn(s, gnb[...], bnb[...])
    z = jax.nn.relu(xn_ref[...] + aggr + e_ref[...])
    z = jax.nn.relu(_bn(_mm(z, w1[...]) + b1[...], g1[...], bb1[...]))
    x_out[...] = jax.nn.relu(_bn(_mm(z, w2[...]) + b2[...], g2[...], bb2[...]))


def _row(v):
    return v.reshape(1, -1)


def kernel(node_attr, edge_index, edge_attr, params):
    row = edge_index[0]
    col = edge_index[1]
    pad = E_PAD - E
    row_p = jnp.concatenate([row, jnp.full((pad,), N, jnp.int32)])
    col_p = jnp.concatenate([col, jnp.zeros((pad,), jnp.int32)])
    ea_aug = jnp.concatenate(
        [edge_attr.astype(jnp.float32),
         jnp.ones((E, 1), jnp.float32),
         jnp.zeros((E, EW - EDGE_DIM - 1), jnp.float32)], axis=1)
    ea_aug = jnp.concatenate([ea_aug, jnp.zeros((pad, EW), jnp.float32)], axis=0)
    zeros128 = jnp.zeros((N_PAD, EMBED), jnp.float32)
    zeros32 = jnp.zeros((N_PAD, EW), jnp.float32)

    p0 = params["mlp0"]
    L1, L2 = params["layers"][0], params["layers"][1]

    h1, xn1 = pl.pallas_call(
        _dense1_body,
        out_shape=(jax.ShapeDtypeStruct((N, EMBED), jnp.float32),
                   jax.ShapeDtypeStruct((N, EMBED), jnp.float32)),
    )(node_attr, p0["W"], _row(p0["b"]), _row(p0["g"]), _row(p0["beta"]),
      L1["Wn"], _row(L1["bn"]), _row(L1["g_node"]), _row(L1["b_node"]),
      L1["Wh"], _row(L1["bh"]))

    sparts = _edge_sum(ea_aug, row_p, zeros32)
    parts1 = _spmm(h1, col_p, row_p, zeros128)

    z2, e2 = pl.pallas_call(
        _mid_a_body,
        out_shape=(jax.ShapeDtypeStruct((N, EMBED), jnp.float32),
                   jax.ShapeDtypeStruct((N, EMBED), jnp.float32)),
    )(xn1, parts1, sparts,
      L1["We"], _row(L1["be"]), _row(L1["g_edge"]), _row(L1["b_edge"]),
      _row(L1["g_nb"]), _row(L1["b_nb"]),
      L1["W1"], _row(L1["b1"]), _row(L1["g1"]), _row(L1["bb1"]),
      L2["We"], _row(L2["be"]), _row(L2["g_edge"]), _row(L2["b_edge"]))

    h2, xn2 = pl.pallas_call(
        _mid_b_body,
        out_shape=(jax.ShapeDtypeStruct((N, EMBED), jnp.float32),
                   jax.ShapeDtypeStruct((N, EMBED), jnp.float32)),
    )(z2, L1["W2"], _row(L1["b2"]), _row(L1["g2"]), _row(L1["bb2"]),
      L2["Wn"], _row(L2["bn"]), _row(L2["g_node"]), _row(L2["b_node"]),
      L2["Wh"], _row(L2["bh"]))

    parts2 = _spmm(h2, col_p, row_p, zeros128)

    x_out = pl.pallas_call(
        _dense_final_body,
        out_shape=jax.ShapeDtypeStruct((N, EMBED), jnp.float32),
    )(xn2, e2, parts2,
      _row(L2["g_nb"]), _row(L2["b_nb"]),
      L2["W1"], _row(L2["b1"]), _row(L2["g1"]), _row(L2["bb1"]),
      L2["W2"], _row(L2["b2"]), _row(L2["g2"]), _row(L2["bb2"]))

    return x_out


# speedup vs baseline: 2.6491x; 1.2201x over previous
import functools

import jax
import jax.numpy as jnp
from jax import lax
from jax.experimental import pallas as pl
from jax.experimental.pallas import tpu as pltpu
from jax.experimental.pallas import tpu_sc as plsc

N = 10000
E = 320000
NODE_DIM = 128
EDGE_DIM = 16
EMBED = 128
EPS = 1e-5

NC = 2
NS = 16
NW = NC * NS
CHUNK = 128
E_PAD = 327680
EDGES_PER_TILE = E_PAD // NW
CHUNKS_PER_TILE = EDGES_PER_TILE // CHUNK
N_PAD = 10240
ROWS_PER_TILE = N_PAD // NS
EW = 32


def _sc_mesh():
    return plsc.VectorSubcoreMesh(core_axis_name="c", subcore_axis_name="s")


def _edge_sum_body(ea_hbm, row2_hbm, zeros_hbm, out_hbm,
                   row_v, buf0, buf1, acc_sh, sem0, sem1):
    ci = lax.axis_index("c")
    si = lax.axis_index("s")
    wid = ci * NS + si
    r0 = si * ROWS_PER_TILE
    pltpu.sync_copy(zeros_hbm.at[pl.ds(r0, ROWS_PER_TILE)],
                    acc_sh.at[pl.ds(r0, ROWS_PER_TILE)])
    cbase = wid * CHUNKS_PER_TILE
    pltpu.sync_copy(row2_hbm.at[pl.ds(cbase, CHUNKS_PER_TILE)], row_v)
    plsc.subcore_barrier()
    ebase = wid * EDGES_PER_TILE

    def start(j, buf, sem):
        return pltpu.async_copy(ea_hbm.at[pl.ds(ebase + j * CHUNK, CHUNK)], buf, sem)

    start(0, buf0, sem0)

    @pl.loop(0, CHUNKS_PER_TILE, step=2)
    def _(j):
        start(j + 1, buf1, sem1)
        pltpu.make_async_copy(ea_hbm.at[pl.ds(0, CHUNK)], buf0, sem0).wait()
        pltpu.sync_copy(buf0, acc_sh.at[row_v.at[j]], add=True)

        @pl.when(j + 2 < CHUNKS_PER_TILE)
        def _():
            start(j + 2, buf0, sem0)

        pltpu.make_async_copy(ea_hbm.at[pl.ds(0, CHUNK)], buf1, sem1).wait()
        pltpu.sync_copy(buf1, acc_sh.at[row_v.at[j + 1]], add=True)

    plsc.subcore_barrier()
    pltpu.sync_copy(acc_sh.at[pl.ds(r0, ROWS_PER_TILE)],
                    out_hbm.at[ci, pl.ds(r0, ROWS_PER_TILE)])


_edge_sum = pl.kernel(
    _edge_sum_body,
    out_type=jax.ShapeDtypeStruct((NC, N_PAD, EW), jnp.float32),
    mesh=_sc_mesh(),
    scratch_types=[
        pltpu.VMEM((CHUNKS_PER_TILE, CHUNK), jnp.int32),
        pltpu.VMEM((CHUNK, EW), jnp.float32),
        pltpu.VMEM((CHUNK, EW), jnp.float32),
        pltpu.VMEM_SHARED((N_PAD, EW), jnp.float32),
        pltpu.SemaphoreType.DMA,
        pltpu.SemaphoreType.DMA,
    ],
    compiler_params=pltpu.CompilerParams(use_tc_tiling_on_sc=False),
)


def _spmm_body(h_hbm, col2_hbm, row2_hbm, zeros_hbm, out_hbm,
               col_v, row_v, buf0, buf1, acc_sh, sem0, sem1):
    ci = lax.axis_index("c")
    si = lax.axis_index("s")
    wid = ci * NS + si
    r0 = si * ROWS_PER_TILE
    pltpu.sync_copy(zeros_hbm.at[pl.ds(r0, ROWS_PER_TILE)],
                    acc_sh.at[pl.ds(r0, ROWS_PER_TILE)])
    cbase = wid * CHUNKS_PER_TILE
    plsc.subcore_barrier()

    def start(j, buf, sem):
        return pltpu.async_copy(h_hbm.at[col_v.at[j]], buf, sem)

    for half in range(2):
        hc = CHUNKS_PER_TILE // 2
        pltpu.sync_copy(col2_hbm.at[pl.ds(cbase + half * hc, hc)], col_v)
        pltpu.sync_copy(row2_hbm.at[pl.ds(cbase + half * hc, hc)], row_v)
        start(0, buf0, sem0)

        @pl.loop(0, hc, step=2)
        def _(j):
            start(j + 1, buf1, sem1)
            pltpu.make_async_copy(h_hbm.at[pl.ds(0, CHUNK)], buf0, sem0).wait()
            pltpu.sync_copy(buf0, acc_sh.at[row_v.at[j]], add=True)

            @pl.when(j + 2 < hc)
            def _():
                start(j + 2, buf0, sem0)

            pltpu.make_async_copy(h_hbm.at[pl.ds(0, CHUNK)], buf1, sem1).wait()
            pltpu.sync_copy(buf1, acc_sh.at[row_v.at[j + 1]], add=True)

    plsc.subcore_barrier()
    pltpu.sync_copy(acc_sh.at[pl.ds(r0, ROWS_PER_TILE)],
                    out_hbm.at[ci, pl.ds(r0, ROWS_PER_TILE)])


_spmm = pl.kernel(
    _spmm_body,
    out_type=jax.ShapeDtypeStruct((NC, N_PAD, EMBED), jnp.float32),
    mesh=_sc_mesh(),
    scratch_types=[
        pltpu.VMEM((CHUNKS_PER_TILE // 2, CHUNK), jnp.int32),
        pltpu.VMEM((CHUNKS_PER_TILE // 2, CHUNK), jnp.int32),
        pltpu.VMEM((CHUNK, EMBED), jnp.float32),
        pltpu.VMEM((CHUNK, EMBED), jnp.float32),
        pltpu.VMEM_SHARED((N_PAD, EMBED), jnp.float32),
        pltpu.SemaphoreType.DMA,
        pltpu.SemaphoreType.DMA,
    ],
)


def _bn(y, g, b):
    m = jnp.mean(y, axis=0, keepdims=True)
    v = jnp.mean((y - m) ** 2, axis=0, keepdims=True)
    return g * (y - m) / jnp.sqrt(v + EPS) + b


def _mm(x, w):
    return jnp.dot(x, w, preferred_element_type=jnp.float32)


def _dense1_body(node_ref, w0, b0, g0, beta0, wn, bn_, gnode, bnode, wh, bh,
                 h1_out, xn1_out):
    x = jax.nn.relu(_bn(_mm(node_ref[...], w0[...]) + b0[...], g0[...], beta0[...]))
    h1_out[...] = _mm(x, wh[...]) + bh[...]
    xn1_out[...] = _bn(_mm(x, wn[...]) + bn_[...], gnode[...], bnode[...])


def _mid_a_body(xn_ref, part_ref, spart_ref,
                we, be, gedge, bedge, gnb, bnb, w1, b1, g1, bb1,
                we_n, be_n, gedge_n, bedge_n,
                z2_out, e2_out):
    s = part_ref[0, :N, :] + part_ref[1, :N, :]
    aggr = _bn(s, gnb[...], bnb[...])
    s32 = spart_ref[0, :N, :] + spart_ref[1, :N, :]
    se = s32[:, :EDGE_DIM]
    cnt = s32[:, EDGE_DIM:EDGE_DIM + 1]
    e = _bn(_mm(se, we[...]) + cnt * be[...], gedge[...], bedge[...])
    z = jax.nn.relu(xn_ref[...] + aggr + e)
    z2_out[...] = jax.nn.relu(_bn(_mm(z, w1[...]) + b1[...], g1[...], bb1[...]))
    e2_out[...] = _bn(_mm(se, we_n[...]) + cnt * be_n[...], gedge_n[...], bedge_n[...])


def _mid_b_body(z2_ref, w2, b2, g2, bb2, wn, bn_, gnode, bnode, wh, bh,
                h2_out, xn2_out):
    x = jax.nn.relu(_bn(_mm(z2_ref[...], w2[...]) + b2[...], g2[...], bb2[...]))
    h2_out[...] = _mm(x, wh[...]) + bh[...]
    xn2_out[...] = _bn(_mm(x, wn[...]) + bn_[...], gnode[...], bnode[...])


def _dense_final_body(xn_ref, e_ref, part_ref,
                      gnb, bnb, w1, b1, g1, bb1, w2, b2, g2, bb2,
                      x_out):
    s = part_ref[0, :N, :] + part_ref[1, :N, :]
    aggr = _bn(s, gnb[...], bnb[...])
    z = jax.nn.relu(xn_ref[...] + aggr + e_ref[...])
    z = jax.nn.relu(_bn(_mm(z, w1[...]) + b1[...], g1[...], bb1[...]))
    x_out[...] = jax.nn.relu(_bn(_mm(z, w2[...]) + b2[...], g2[...], bb2[...]))


def _row(v):
    return v.reshape(1, -1)


def kernel(node_attr, edge_index, edge_attr, params):
    row = edge_index[0]
    col = edge_index[1]
    pad = E_PAD - E
    row_pad_tgt = N + (jnp.arange(pad, dtype=jnp.int32) % (N_PAD - N))
    row_p = jnp.concatenate([row, row_pad_tgt]).reshape(E_PAD // CHUNK, CHUNK)
    col_p = jnp.concatenate([col, jnp.zeros((pad,), jnp.int32)]).reshape(
        E_PAD // CHUNK, CHUNK)
    ea_aug = jnp.concatenate(
        [edge_attr.astype(jnp.float32),
         jnp.ones((E, 1), jnp.float32),
         jnp.zeros((E, EW - EDGE_DIM - 1), jnp.float32)], axis=1)
    ea_aug = jnp.concatenate([ea_aug, jnp.zeros((pad, EW), jnp.float32)], axis=0)
    zeros128 = jnp.zeros((N_PAD, EMBED), jnp.float32)
    zeros32 = jnp.zeros((N_PAD, EW), jnp.float32)

    p0 = params["mlp0"]
    L1, L2 = params["layers"][0], params["layers"][1]

    h1, xn1 = pl.pallas_call(
        _dense1_body,
        out_shape=(jax.ShapeDtypeStruct((N, EMBED), jnp.float32),
                   jax.ShapeDtypeStruct((N, EMBED), jnp.float32)),
    )(node_attr, p0["W"], _row(p0["b"]), _row(p0["g"]), _row(p0["beta"]),
      L1["Wn"], _row(L1["bn"]), _row(L1["g_node"]), _row(L1["b_node"]),
      L1["Wh"], _row(L1["bh"]))

    sparts = _edge_sum(ea_aug, row_p, zeros32)
    parts1 = _spmm(h1, col_p, row_p, zeros128)

    z2, e2 = pl.pallas_call(
        _mid_a_body,
        out_shape=(jax.ShapeDtypeStruct((N, EMBED), jnp.float32),
                   jax.ShapeDtypeStruct((N, EMBED), jnp.float32)),
    )(xn1, parts1, sparts,
      L1["We"], _row(L1["be"]), _row(L1["g_edge"]), _row(L1["b_edge"]),
      _row(L1["g_nb"]), _row(L1["b_nb"]),
      L1["W1"], _row(L1["b1"]), _row(L1["g1"]), _row(L1["bb1"]),
      L2["We"], _row(L2["be"]), _row(L2["g_edge"]), _row(L2["b_edge"]))

    h2, xn2 = pl.pallas_call(
        _mid_b_body,
        out_shape=(jax.ShapeDtypeStruct((N, EMBED), jnp.float32),
                   jax.ShapeDtypeStruct((N, EMBED), jnp.float32)),
    )(z2, L1["W2"], _row(L1["b2"]), _row(L1["g2"]), _row(L1["bb2"]),
      L2["Wn"], _row(L2["bn"]), _row(L2["g_node"]), _row(L2["b_node"]),
      L2["Wh"], _row(L2["bh"]))

    parts2 = _spmm(h2, col_p, row_p, zeros128)

    x_out = pl.pallas_call(
        _dense_final_body,
        out_shape=jax.ShapeDtypeStruct((N, EMBED), jnp.float32),
    )(xn2, e2, parts2,
      _row(L2["g_nb"]), _row(L2["b_nb"]),
      L2["W1"], _row(L2["b1"]), _row(L2["g1"]), _row(L2["bb1"]),
      L2["W2"], _row(L2["b2"]), _row(L2["g2"]), _row(L2["bb2"]))

    return x_out


# speedup vs baseline: 5.8263x; 2.1994x over previous
import functools

import jax
import jax.numpy as jnp
from jax import lax
from jax.experimental import pallas as pl
from jax.experimental.pallas import tpu as pltpu
from jax.experimental.pallas import tpu_sc as plsc

N = 10000
E = 320000
NODE_DIM = 128
EDGE_DIM = 16
EMBED = 128
EPS = 1e-5

NC = 2
NS = 16
NW = NC * NS
CHUNK = 128
E_PAD = 327680
EDGES_PER_TILE = E_PAD // NW
CHUNKS_PER_TILE = EDGES_PER_TILE // CHUNK
N_PAD = 10240
ROWS_PER_TILE = N_PAD // NS
EW = 32


def _sc_mesh():
    return plsc.VectorSubcoreMesh(core_axis_name="c", subcore_axis_name="s")


def _edge_sum_body(ea_hbm, row2_hbm, zeros_hbm, out_hbm,
                   row_v, buf0, buf1, acc_sh, sem0, sem1):
    ci = lax.axis_index("c")
    si = lax.axis_index("s")
    wid = ci * NS + si
    r0 = si * ROWS_PER_TILE
    pltpu.sync_copy(zeros_hbm.at[pl.ds(r0, ROWS_PER_TILE)],
                    acc_sh.at[pl.ds(r0, ROWS_PER_TILE)])
    cbase = wid * CHUNKS_PER_TILE
    pltpu.sync_copy(row2_hbm.at[pl.ds(cbase, CHUNKS_PER_TILE)], row_v)
    plsc.subcore_barrier()
    ebase = wid * EDGES_PER_TILE

    def start(j, buf, sem):
        return pltpu.async_copy(ea_hbm.at[pl.ds(ebase + j * CHUNK, CHUNK)], buf, sem)

    start(0, buf0, sem0)

    @pl.loop(0, CHUNKS_PER_TILE, step=2)
    def _(j):
        start(j + 1, buf1, sem1)
        pltpu.make_async_copy(ea_hbm.at[pl.ds(0, CHUNK)], buf0, sem0).wait()
        pltpu.sync_copy(buf0, acc_sh.at[row_v.at[j]], add=True)

        @pl.when(j + 2 < CHUNKS_PER_TILE)
        def _():
            start(j + 2, buf0, sem0)

        pltpu.make_async_copy(ea_hbm.at[pl.ds(0, CHUNK)], buf1, sem1).wait()
        pltpu.sync_copy(buf1, acc_sh.at[row_v.at[j + 1]], add=True)

    plsc.subcore_barrier()
    pltpu.sync_copy(acc_sh.at[pl.ds(r0, ROWS_PER_TILE)],
                    out_hbm.at[ci, pl.ds(r0, ROWS_PER_TILE)])


_edge_sum = pl.kernel(
    _edge_sum_body,
    out_type=jax.ShapeDtypeStruct((NC, N_PAD, EW), jnp.float32),
    mesh=_sc_mesh(),
    scratch_types=[
        pltpu.VMEM((CHUNKS_PER_TILE, CHUNK), jnp.int32),
        pltpu.VMEM((CHUNK, EW), jnp.float32),
        pltpu.VMEM((CHUNK, EW), jnp.float32),
        pltpu.VMEM_SHARED((N_PAD, EW), jnp.float32),
        pltpu.SemaphoreType.DMA,
        pltpu.SemaphoreType.DMA,
    ],
    compiler_params=pltpu.CompilerParams(use_tc_tiling_on_sc=False),
)


def _spmm_body(h_hbm, col2_hbm, row2_hbm, zeros_hbm, out_hbm,
               col_v, row_v, buf0, buf1, acc_sh, sem0, sem1):
    ci = lax.axis_index("c")
    si = lax.axis_index("s")
    wid = ci * NS + si
    r0 = si * ROWS_PER_TILE
    pltpu.sync_copy(zeros_hbm.at[pl.ds(r0, ROWS_PER_TILE)],
                    acc_sh.at[pl.ds(r0, ROWS_PER_TILE)])
    cbase = wid * CHUNKS_PER_TILE
    plsc.subcore_barrier()

    def start(j, buf, sem):
        return pltpu.async_copy(h_hbm.at[col_v.at[j]], buf, sem)

    for half in range(2):
        hc = CHUNKS_PER_TILE // 2
        pltpu.sync_copy(col2_hbm.at[pl.ds(cbase + half * hc, hc)], col_v)
        pltpu.sync_copy(row2_hbm.at[pl.ds(cbase + half * hc, hc)], row_v)
        start(0, buf0, sem0)

        @pl.loop(0, hc, step=2)
        def _(j):
            start(j + 1, buf1, sem1)
            pltpu.make_async_copy(h_hbm.at[pl.ds(0, CHUNK)], buf0, sem0).wait()
            pltpu.sync_copy(buf0, acc_sh.at[row_v.at[j]], add=True)

            @pl.when(j + 2 < hc)
            def _():
                start(j + 2, buf0, sem0)

            pltpu.make_async_copy(h_hbm.at[pl.ds(0, CHUNK)], buf1, sem1).wait()
            pltpu.sync_copy(buf1, acc_sh.at[row_v.at[j + 1]], add=True)

    plsc.subcore_barrier()
    pltpu.sync_copy(acc_sh.at[pl.ds(r0, ROWS_PER_TILE)],
                    out_hbm.at[ci, pl.ds(r0, ROWS_PER_TILE)])


_spmm = pl.kernel(
    _spmm_body,
    out_type=jax.ShapeDtypeStruct((NC, N_PAD, EMBED), jnp.float32),
    mesh=_sc_mesh(),
    scratch_types=[
        pltpu.VMEM((CHUNKS_PER_TILE // 2, CHUNK), jnp.int32),
        pltpu.VMEM((CHUNKS_PER_TILE // 2, CHUNK), jnp.int32),
        pltpu.VMEM((CHUNK, EMBED), jnp.float32),
        pltpu.VMEM((CHUNK, EMBED), jnp.float32),
        pltpu.VMEM_SHARED((N_PAD, EMBED), jnp.float32),
        pltpu.SemaphoreType.DMA,
        pltpu.SemaphoreType.DMA,
    ],
)


def _bn(y, g, b):
    m = jnp.mean(y, axis=0, keepdims=True)
    v = jnp.mean((y - m) ** 2, axis=0, keepdims=True)
    return g * (y - m) / jnp.sqrt(v + EPS) + b


def _mm(x, w):
    return jnp.dot(x, w, preferred_element_type=jnp.float32)


def _dense1_body(node_ref, w0, b0, g0, beta0, wn, bn_, gnode, bnode, wh, bh,
                 h1_out, xn1_out):
    x = jax.nn.relu(_bn(_mm(node_ref[...], w0[...]) + b0[...], g0[...], beta0[...]))
    h1_out[...] = _mm(x, wh[...]) + bh[...]
    xn1_out[...] = _bn(_mm(x, wn[...]) + bn_[...], gnode[...], bnode[...])


def _mid_a_body(xn_ref, part_ref, spart_ref,
                we, be, gedge, bedge, gnb, bnb, w1, b1, g1, bb1,
                we_n, be_n, gedge_n, bedge_n,
                z2_out, e2_out):
    s = part_ref[0, :N, :] + part_ref[1, :N, :]
    aggr = _bn(s, gnb[...], bnb[...])
    s32 = spart_ref[0, :N, :] + spart_ref[1, :N, :]
    se = s32[:, :EDGE_DIM]
    cnt = s32[:, EDGE_DIM:EDGE_DIM + 1]
    e = _bn(_mm(se, we[...]) + cnt * be[...], gedge[...], bedge[...])
    z = jax.nn.relu(xn_ref[...] + aggr + e)
    z2_out[...] = jax.nn.relu(_bn(_mm(z, w1[...]) + b1[...], g1[...], bb1[...]))
    e2_out[...] = _bn(_mm(se, we_n[...]) + cnt * be_n[...], gedge_n[...], bedge_n[...])


def _mid_b_body(z2_ref, w2, b2, g2, bb2, wn, bn_, gnode, bnode, wh, bh,
                h2_out, xn2_out):
    x = jax.nn.relu(_bn(_mm(z2_ref[...], w2[...]) + b2[...], g2[...], bb2[...]))
    h2_out[...] = _mm(x, wh[...]) + bh[...]
    xn2_out[...] = _bn(_mm(x, wn[...]) + bn_[...], gnode[...], bnode[...])


def _dense_final_body(xn_ref, e_ref, part_ref,
                      gnb, bnb, w1, b1, g1, bb1, w2, b2, g2, bb2,
                      x_out):
    s = part_ref[0, :N, :] + part_ref[1, :N, :]
    aggr = _bn(s, gnb[...], bnb[...])
    z = jax.nn.relu(xn_ref[...] + aggr + e_ref[...])
    z = jax.nn.relu(_bn(_mm(z, w1[...]) + b1[...], g1[...], bb1[...]))
    x_out[...] = jax.nn.relu(_bn(_mm(z, w2[...]) + b2[...], g2[...], bb2[...]))


def _row(v):
    return v.reshape(1, -1)


def kernel(node_attr, edge_index, edge_attr, params):
    row = edge_index[0]
    col = edge_index[1]
    pad = E_PAD - E
    row_pad_tgt = N + (jnp.arange(pad, dtype=jnp.int32) % (N_PAD - N))
    col_pad_tgt = jnp.arange(pad, dtype=jnp.int32) % N
    row_p = jnp.concatenate([row, row_pad_tgt]).reshape(E_PAD // CHUNK, CHUNK)
    col_p = jnp.concatenate([col, col_pad_tgt]).reshape(E_PAD // CHUNK, CHUNK)
    ea_aug = jnp.concatenate(
        [edge_attr.astype(jnp.float32),
         jnp.ones((E, 1), jnp.float32),
         jnp.zeros((E, EW - EDGE_DIM - 1), jnp.float32)], axis=1)
    ea_aug = jnp.concatenate([ea_aug, jnp.zeros((pad, EW), jnp.float32)], axis=0)
    zeros128 = jnp.zeros((N_PAD, EMBED), jnp.float32)
    zeros32 = jnp.zeros((N_PAD, EW), jnp.float32)

    p0 = params["mlp0"]
    L1, L2 = params["layers"][0], params["layers"][1]

    h1, xn1 = pl.pallas_call(
        _dense1_body,
        out_shape=(jax.ShapeDtypeStruct((N, EMBED), jnp.float32),
                   jax.ShapeDtypeStruct((N, EMBED), jnp.float32)),
    )(node_attr, p0["W"], _row(p0["b"]), _row(p0["g"]), _row(p0["beta"]),
      L1["Wn"], _row(L1["bn"]), _row(L1["g_node"]), _row(L1["b_node"]),
      L1["Wh"], _row(L1["bh"]))

    sparts = _edge_sum(ea_aug, row_p, zeros32)
    parts1 = _spmm(h1, col_p, row_p, zeros128)

    z2, e2 = pl.pallas_call(
        _mid_a_body,
        out_shape=(jax.ShapeDtypeStruct((N, EMBED), jnp.float32),
                   jax.ShapeDtypeStruct((N, EMBED), jnp.float32)),
    )(xn1, parts1, sparts,
      L1["We"], _row(L1["be"]), _row(L1["g_edge"]), _row(L1["b_edge"]),
      _row(L1["g_nb"]), _row(L1["b_nb"]),
      L1["W1"], _row(L1["b1"]), _row(L1["g1"]), _row(L1["bb1"]),
      L2["We"], _row(L2["be"]), _row(L2["g_edge"]), _row(L2["b_edge"]))

    h2, xn2 = pl.pallas_call(
        _mid_b_body,
        out_shape=(jax.ShapeDtypeStruct((N, EMBED), jnp.float32),
                   jax.ShapeDtypeStruct((N, EMBED), jnp.float32)),
    )(z2, L1["W2"], _row(L1["b2"]), _row(L1["g2"]), _row(L1["bb2"]),
      L2["Wn"], _row(L2["bn"]), _row(L2["g_node"]), _row(L2["b_node"]),
      L2["Wh"], _row(L2["bh"]))

    parts2 = _spmm(h2, col_p, row_p, zeros128)

    x_out = pl.pallas_call(
        _dense_final_body,
        out_shape=jax.ShapeDtypeStruct((N, EMBED), jnp.float32),
    )(xn2, e2, parts2,
      _row(L2["g_nb"]), _row(L2["b_nb"]),
      L2["W1"], _row(L2["b1"]), _row(L2["g1"]), _row(L2["bb1"]),
      L2["W2"], _row(L2["b2"]), _row(L2["g2"]), _row(L2["bb2"]))

    return x_out


# speedup vs baseline: 7.4514x; 1.2789x over previous
import functools

import jax
import jax.numpy as jnp
from jax import lax
from jax.experimental import pallas as pl
from jax.experimental.pallas import tpu as pltpu
from jax.experimental.pallas import tpu_sc as plsc

N = 10000
E = 320000
NODE_DIM = 128
EDGE_DIM = 16
EMBED = 128
EPS = 1e-5

NC = 2
NS = 16
NW = NC * NS
CHUNK = 128
E_PAD = 327680
EDGES_PER_TILE = E_PAD // NW
CHUNKS_PER_TILE = EDGES_PER_TILE // CHUNK
N_PAD = 10240
ROWS_PER_TILE = N_PAD // NS
EW = 32


def _sc_mesh():
    return plsc.VectorSubcoreMesh(core_axis_name="c", subcore_axis_name="s")


def _edge_sum_body(ea_hbm, row2_hbm, ones_hbm, zeros_hbm, out_hbm, cnt_hbm,
                   row_v, ones_v, buf0, buf1, acc_sh, cnt_sh, sem0, sem1):
    ci = lax.axis_index("c")
    si = lax.axis_index("s")
    wid = ci * NS + si
    r0 = si * ROWS_PER_TILE
    pltpu.sync_copy(zeros_hbm.at[pl.ds(r0, ROWS_PER_TILE)],
                    acc_sh.at[pl.ds(r0, ROWS_PER_TILE)])
    pltpu.sync_copy(zeros_hbm.at[pl.ds(r0, ROWS_PER_TILE)],
                    cnt_sh.at[pl.ds(r0, ROWS_PER_TILE)])
    pltpu.sync_copy(ones_hbm, ones_v)
    cbase = wid * CHUNKS_PER_TILE
    pltpu.sync_copy(row2_hbm.at[pl.ds(cbase, CHUNKS_PER_TILE)], row_v)
    plsc.subcore_barrier()
    ebase = wid * EDGES_PER_TILE
    ne = ea_hbm.shape[0]

    def start(j, buf, sem):
        off = ebase + j * CHUNK
        off = jnp.where(off + CHUNK <= ne, off, 0)
        return pltpu.async_copy(ea_hbm.at[pl.ds(off, CHUNK)], buf, sem)

    start(0, buf0, sem0)

    @pl.loop(0, CHUNKS_PER_TILE, step=2)
    def _(j):
        start(j + 1, buf1, sem1)
        pltpu.make_async_copy(ea_hbm.at[pl.ds(0, CHUNK)], buf0, sem0).wait()
        pltpu.sync_copy(buf0, acc_sh.at[row_v.at[j]], add=True)
        pltpu.sync_copy(ones_v, cnt_sh.at[row_v.at[j]], add=True)

        @pl.when(j + 2 < CHUNKS_PER_TILE)
        def _():
            start(j + 2, buf0, sem0)

        pltpu.make_async_copy(ea_hbm.at[pl.ds(0, CHUNK)], buf1, sem1).wait()
        pltpu.sync_copy(buf1, acc_sh.at[row_v.at[j + 1]], add=True)
        pltpu.sync_copy(ones_v, cnt_sh.at[row_v.at[j + 1]], add=True)

    plsc.subcore_barrier()
    pltpu.sync_copy(acc_sh.at[pl.ds(r0, ROWS_PER_TILE)],
                    out_hbm.at[ci, pl.ds(r0, ROWS_PER_TILE)])
    pltpu.sync_copy(cnt_sh.at[pl.ds(r0, ROWS_PER_TILE)],
                    cnt_hbm.at[ci, pl.ds(r0, ROWS_PER_TILE)])


_edge_sum = pl.kernel(
    _edge_sum_body,
    out_type=(jax.ShapeDtypeStruct((NC, N_PAD, EDGE_DIM), jnp.float32),
              jax.ShapeDtypeStruct((NC, N_PAD, EDGE_DIM), jnp.float32)),
    mesh=_sc_mesh(),
    scratch_types=[
        pltpu.VMEM((CHUNKS_PER_TILE, CHUNK), jnp.int32),
        pltpu.VMEM((CHUNK, EDGE_DIM), jnp.float32),
        pltpu.VMEM((CHUNK, EDGE_DIM), jnp.float32),
        pltpu.VMEM((CHUNK, EDGE_DIM), jnp.float32),
        pltpu.VMEM_SHARED((N_PAD, EDGE_DIM), jnp.float32),
        pltpu.VMEM_SHARED((N_PAD, EDGE_DIM), jnp.float32),
        pltpu.SemaphoreType.DMA,
        pltpu.SemaphoreType.DMA,
    ],
    compiler_params=pltpu.CompilerParams(use_tc_tiling_on_sc=False),
)


def _spmm_body(h_hbm, col2_hbm, row2_hbm, zeros_hbm, out_hbm,
               col_v, row_v, buf0, buf1, acc_sh, sem0, sem1):
    ci = lax.axis_index("c")
    si = lax.axis_index("s")
    wid = ci * NS + si
    r0 = si * ROWS_PER_TILE
    pltpu.sync_copy(zeros_hbm.at[pl.ds(r0, ROWS_PER_TILE)],
                    acc_sh.at[pl.ds(r0, ROWS_PER_TILE)])
    cbase = wid * CHUNKS_PER_TILE
    plsc.subcore_barrier()

    def start(j, buf, sem):
        return pltpu.async_copy(h_hbm.at[col_v.at[j]], buf, sem)

    for half in range(2):
        hc = CHUNKS_PER_TILE // 2
        pltpu.sync_copy(col2_hbm.at[pl.ds(cbase + half * hc, hc)], col_v)
        pltpu.sync_copy(row2_hbm.at[pl.ds(cbase + half * hc, hc)], row_v)
        start(0, buf0, sem0)

        @pl.loop(0, hc, step=2)
        def _(j):
            start(j + 1, buf1, sem1)
            pltpu.make_async_copy(h_hbm.at[pl.ds(0, CHUNK)], buf0, sem0).wait()
            pltpu.sync_copy(buf0, acc_sh.at[row_v.at[j]], add=True)

            @pl.when(j + 2 < hc)
            def _():
                start(j + 2, buf0, sem0)

            pltpu.make_async_copy(h_hbm.at[pl.ds(0, CHUNK)], buf1, sem1).wait()
            pltpu.sync_copy(buf1, acc_sh.at[row_v.at[j + 1]], add=True)

    plsc.subcore_barrier()
    pltpu.sync_copy(acc_sh.at[pl.ds(r0, ROWS_PER_TILE)],
                    out_hbm.at[ci, pl.ds(r0, ROWS_PER_TILE)])


_spmm = pl.kernel(
    _spmm_body,
    out_type=jax.ShapeDtypeStruct((NC, N_PAD, EMBED), jnp.float32),
    mesh=_sc_mesh(),
    scratch_types=[
        pltpu.VMEM((CHUNKS_PER_TILE // 2, CHUNK), jnp.int32),
        pltpu.VMEM((CHUNKS_PER_TILE // 2, CHUNK), jnp.int32),
        pltpu.VMEM((CHUNK, EMBED), jnp.float32),
        pltpu.VMEM((CHUNK, EMBED), jnp.float32),
        pltpu.VMEM_SHARED((N_PAD, EMBED), jnp.float32),
        pltpu.SemaphoreType.DMA,
        pltpu.SemaphoreType.DMA,
    ],
)


def _bn(y, g, b):
    m = jnp.mean(y, axis=0, keepdims=True)
    v = jnp.mean((y - m) ** 2, axis=0, keepdims=True)
    return g * (y - m) / jnp.sqrt(v + EPS) + b


def _mm(x, w):
    return jnp.dot(x, w, preferred_element_type=jnp.float32)


def _dense1_body(node_ref, w0, b0, g0, beta0, wn, bn_, gnode, bnode, wh, bh,
                 h1_out, xn1_out):
    x = jax.nn.relu(_bn(_mm(node_ref[...], w0[...]) + b0[...], g0[...], beta0[...]))
    h1_out[...] = _mm(x, wh[...]) + bh[...]
    xn1_out[...] = _bn(_mm(x, wn[...]) + bn_[...], gnode[...], bnode[...])


def _mid_a_body(xn_ref, part_ref, spart_ref, cpart_ref,
                we, be, gedge, bedge, gnb, bnb, w1, b1, g1, bb1,
                we_n, be_n, gedge_n, bedge_n,
                z2_out, e2_out):
    s = part_ref[0, :N, :] + part_ref[1, :N, :]
    aggr = _bn(s, gnb[...], bnb[...])
    se = spart_ref[0, :N, :] + spart_ref[1, :N, :]
    cnt = cpart_ref[0, :N, :1] + cpart_ref[1, :N, :1]
    e = _bn(_mm(se, we[...]) + cnt * be[...], gedge[...], bedge[...])
    z = jax.nn.relu(xn_ref[...] + aggr + e)
    z2_out[...] = jax.nn.relu(_bn(_mm(z, w1[...]) + b1[...], g1[...], bb1[...]))
    e2_out[...] = _bn(_mm(se, we_n[...]) + cnt * be_n[...], gedge_n[...], bedge_n[...])


def _mid_b_body(z2_ref, w2, b2, g2, bb2, wn, bn_, gnode, bnode, wh, bh,
                h2_out, xn2_out):
    x = jax.nn.relu(_bn(_mm(z2_ref[...], w2[...]) + b2[...], g2[...], bb2[...]))
    h2_out[...] = _mm(x, wh[...]) + bh[...]
    xn2_out[...] = _bn(_mm(x, wn[...]) + bn_[...], gnode[...], bnode[...])


def _dense_final_body(xn_ref, e_ref, part_ref,
                      gnb, bnb, w1, b1, g1, bb1, w2, b2, g2, bb2,
                      x_out):
    s = part_ref[0, :N, :] + part_ref[1, :N, :]
    aggr = _bn(s, gnb[...], bnb[...])
    z = jax.nn.relu(xn_ref[...] + aggr + e_ref[...])
    z = jax.nn.relu(_bn(_mm(z, w1[...]) + b1[...], g1[...], bb1[...]))
    x_out[...] = jax.nn.relu(_bn(_mm(z, w2[...]) + b2[...], g2[...], bb2[...]))


def _row(v):
    return v.reshape(1, -1)


_TC_PARAMS = pltpu.CompilerParams(vmem_limit_bytes=117 * 1024 * 1024)


def kernel(node_attr, edge_index, edge_attr, params):
    row = edge_index[0]
    col = edge_index[1]
    pad = E_PAD - E
    row_pad_tgt = N + (jnp.arange(pad, dtype=jnp.int32) % (N_PAD - N))
    col_pad_tgt = jnp.arange(pad, dtype=jnp.int32) % N
    row_p = jnp.concatenate([row, row_pad_tgt]).reshape(E_PAD // CHUNK, CHUNK)
    col_p = jnp.concatenate([col, col_pad_tgt]).reshape(E_PAD // CHUNK, CHUNK)
    zeros128 = jnp.zeros((N_PAD, EMBED), jnp.float32)
    zeros16 = jnp.zeros((N_PAD, EDGE_DIM), jnp.float32)
    ones16 = jnp.ones((CHUNK, EDGE_DIM), jnp.float32)

    p0 = params["mlp0"]
    L1, L2 = params["layers"][0], params["layers"][1]

    h1, xn1 = pl.pallas_call(
        _dense1_body,
        out_shape=(jax.ShapeDtypeStruct((N, EMBED), jnp.float32),
                   jax.ShapeDtypeStruct((N, EMBED), jnp.float32)),
        compiler_params=_TC_PARAMS,
    )(node_attr, p0["W"], _row(p0["b"]), _row(p0["g"]), _row(p0["beta"]),
      L1["Wn"], _row(L1["bn"]), _row(L1["g_node"]), _row(L1["b_node"]),
      L1["Wh"], _row(L1["bh"]))

    sparts, cparts = _edge_sum(edge_attr, row_p, ones16, zeros16)
    parts1 = _spmm(h1, col_p, row_p, zeros128)

    z2, e2 = pl.pallas_call(
        _mid_a_body,
        out_shape=(jax.ShapeDtypeStruct((N, EMBED), jnp.float32),
                   jax.ShapeDtypeStruct((N, EMBED), jnp.float32)),
        compiler_params=_TC_PARAMS,
    )(xn1, parts1, sparts, cparts,
      L1["We"], _row(L1["be"]), _row(L1["g_edge"]), _row(L1["b_edge"]),
      _row(L1["g_nb"]), _row(L1["b_nb"]),
      L1["W1"], _row(L1["b1"]), _row(L1["g1"]), _row(L1["bb1"]),
      L2["We"], _row(L2["be"]), _row(L2["g_edge"]), _row(L2["b_edge"]))

    h2, xn2 = pl.pallas_call(
        _mid_b_body,
        out_shape=(jax.ShapeDtypeStruct((N, EMBED), jnp.float32),
                   jax.ShapeDtypeStruct((N, EMBED), jnp.float32)),
        compiler_params=_TC_PARAMS,
    )(z2, L1["W2"], _row(L1["b2"]), _row(L1["g2"]), _row(L1["bb2"]),
      L2["Wn"], _row(L2["bn"]), _row(L2["g_node"]), _row(L2["b_node"]),
      L2["Wh"], _row(L2["bh"]))

    parts2 = _spmm(h2, col_p, row_p, zeros128)

    x_out = pl.pallas_call(
        _dense_final_body,
        out_shape=jax.ShapeDtypeStruct((N, EMBED), jnp.float32),
        compiler_params=_TC_PARAMS,
    )(xn2, e2, parts2,
      _row(L2["g_nb"]), _row(L2["b_nb"]),
      L2["W1"], _row(L2["b1"]), _row(L2["g1"]), _row(L2["bb1"]),
      L2["W2"], _row(L2["b2"]), _row(L2["g2"]), _row(L2["bb2"]))

    return x_out


# speedup vs baseline: 7.4625x; 1.0015x over previous
import functools

import jax
import jax.numpy as jnp
from jax import lax
from jax.experimental import pallas as pl
from jax.experimental.pallas import tpu as pltpu
from jax.experimental.pallas import tpu_sc as plsc

N = 10000
E = 320000
NODE_DIM = 128
EDGE_DIM = 16
EMBED = 128
EPS = 1e-5

NC = 2
NS = 16
NW = NC * NS
CHUNK = 128
E_PAD = 327680
EDGES_PER_TILE = E_PAD // NW
CHUNKS_PER_TILE = EDGES_PER_TILE // CHUNK
N_PAD = 10240
ROWS_PER_TILE = N_PAD // NS
EW = 32


def _sc_mesh():
    return plsc.VectorSubcoreMesh(core_axis_name="c", subcore_axis_name="s")


def _edge_sum_body(ea_hbm, row2_hbm, ones_hbm, zeros_hbm, out_hbm, cnt_hbm,
                   row_v, ones_v, buf0, buf1, acc_sh, cnt_sh, sem0, sem1):
    ci = lax.axis_index("c")
    si = lax.axis_index("s")
    wid = ci * NS + si
    r0 = si * ROWS_PER_TILE
    pltpu.sync_copy(zeros_hbm.at[pl.ds(r0, ROWS_PER_TILE)],
                    acc_sh.at[pl.ds(r0, ROWS_PER_TILE)])
    pltpu.sync_copy(zeros_hbm.at[pl.ds(r0, ROWS_PER_TILE)],
                    cnt_sh.at[pl.ds(r0, ROWS_PER_TILE)])
    pltpu.sync_copy(ones_hbm, ones_v)
    cbase = wid * CHUNKS_PER_TILE
    pltpu.sync_copy(row2_hbm.at[pl.ds(cbase, CHUNKS_PER_TILE)], row_v)
    plsc.subcore_barrier()
    ebase = wid * EDGES_PER_TILE
    ne = ea_hbm.shape[0]

    def start(j, buf, sem):
        off = ebase + j * CHUNK
        off = jnp.where(off + CHUNK <= ne, off, 0)
        return pltpu.async_copy(ea_hbm.at[pl.ds(off, CHUNK)], buf, sem)

    start(0, buf0, sem0)

    @pl.loop(0, CHUNKS_PER_TILE, step=2)
    def _(j):
        start(j + 1, buf1, sem1)
        pltpu.make_async_copy(ea_hbm.at[pl.ds(0, CHUNK)], buf0, sem0).wait()
        pltpu.sync_copy(buf0, acc_sh.at[row_v.at[j]], add=True)
        pltpu.sync_copy(ones_v, cnt_sh.at[row_v.at[j]], add=True)

        @pl.when(j + 2 < CHUNKS_PER_TILE)
        def _():
            start(j + 2, buf0, sem0)

        pltpu.make_async_copy(ea_hbm.at[pl.ds(0, CHUNK)], buf1, sem1).wait()
        pltpu.sync_copy(buf1, acc_sh.at[row_v.at[j + 1]], add=True)
        pltpu.sync_copy(ones_v, cnt_sh.at[row_v.at[j + 1]], add=True)

    plsc.subcore_barrier()
    pltpu.sync_copy(acc_sh.at[pl.ds(r0, ROWS_PER_TILE)],
                    out_hbm.at[ci, pl.ds(r0, ROWS_PER_TILE)])
    pltpu.sync_copy(cnt_sh.at[pl.ds(r0, ROWS_PER_TILE)],
                    cnt_hbm.at[ci, pl.ds(r0, ROWS_PER_TILE)])


_edge_sum = pl.kernel(
    _edge_sum_body,
    out_type=(jax.ShapeDtypeStruct((NC, N_PAD, EDGE_DIM), jnp.float32),
              jax.ShapeDtypeStruct((NC, N_PAD, EDGE_DIM), jnp.float32)),
    mesh=_sc_mesh(),
    scratch_types=[
        pltpu.VMEM((CHUNKS_PER_TILE, CHUNK), jnp.int32),
        pltpu.VMEM((CHUNK, EDGE_DIM), jnp.float32),
        pltpu.VMEM((CHUNK, EDGE_DIM), jnp.float32),
        pltpu.VMEM((CHUNK, EDGE_DIM), jnp.float32),
        pltpu.VMEM_SHARED((N_PAD, EDGE_DIM), jnp.float32),
        pltpu.VMEM_SHARED((N_PAD, EDGE_DIM), jnp.float32),
        pltpu.SemaphoreType.DMA,
        pltpu.SemaphoreType.DMA,
    ],
    compiler_params=pltpu.CompilerParams(use_tc_tiling_on_sc=False),
)


def _spmm_body(h_hbm, col2_hbm, row2_hbm, zeros_hbm, out_hbm,
               col_v, row_v, buf0, buf1, acc_sh, sem0, sem1):
    ci = lax.axis_index("c")
    si = lax.axis_index("s")
    wid = ci * NS + si
    r0 = si * ROWS_PER_TILE
    pltpu.sync_copy(zeros_hbm.at[pl.ds(r0, ROWS_PER_TILE)],
                    acc_sh.at[pl.ds(r0, ROWS_PER_TILE)])
    cbase = wid * CHUNKS_PER_TILE
    plsc.subcore_barrier()

    def start(j, buf, sem):
        return pltpu.async_copy(h_hbm.at[col_v.at[j]], buf, sem)

    for half in range(2):
        hc = CHUNKS_PER_TILE // 2
        pltpu.sync_copy(col2_hbm.at[pl.ds(cbase + half * hc, hc)], col_v)
        pltpu.sync_copy(row2_hbm.at[pl.ds(cbase + half * hc, hc)], row_v)
        start(0, buf0, sem0)

        @pl.loop(0, hc, step=2)
        def _(j):
            start(j + 1, buf1, sem1)
            pltpu.make_async_copy(h_hbm.at[pl.ds(0, CHUNK)], buf0, sem0).wait()
            pltpu.sync_copy(buf0, acc_sh.at[row_v.at[j]], add=True)

            @pl.when(j + 2 < hc)
            def _():
                start(j + 2, buf0, sem0)

            pltpu.make_async_copy(h_hbm.at[pl.ds(0, CHUNK)], buf1, sem1).wait()
            pltpu.sync_copy(buf1, acc_sh.at[row_v.at[j + 1]], add=True)

    plsc.subcore_barrier()
    pltpu.sync_copy(acc_sh.at[pl.ds(r0, ROWS_PER_TILE)],
                    out_hbm.at[ci, pl.ds(r0, ROWS_PER_TILE)])


_spmm = pl.kernel(
    _spmm_body,
    out_type=jax.ShapeDtypeStruct((NC, N_PAD, EMBED), jnp.float32),
    mesh=_sc_mesh(),
    scratch_types=[
        pltpu.VMEM((CHUNKS_PER_TILE // 2, CHUNK), jnp.int32),
        pltpu.VMEM((CHUNKS_PER_TILE // 2, CHUNK), jnp.int32),
        pltpu.VMEM((CHUNK, EMBED), jnp.float32),
        pltpu.VMEM((CHUNK, EMBED), jnp.float32),
        pltpu.VMEM_SHARED((N_PAD, EMBED), jnp.float32),
        pltpu.SemaphoreType.DMA,
        pltpu.SemaphoreType.DMA,
    ],
)


def _bn(y, g, b):
    m = jnp.mean(y, axis=0, keepdims=True)
    v = jnp.mean((y - m) ** 2, axis=0, keepdims=True)
    return g * (y - m) / jnp.sqrt(v + EPS) + b


def _mm(x, w):
    return jnp.dot(x, w, preferred_element_type=jnp.float32)


def _dense1_body(node_ref, w0, b0, g0, beta0, wn, bn_, gnode, bnode, wh, bh,
                 h1_out, xn1_out):
    x = jax.nn.relu(_bn(_mm(node_ref[...], w0[...]) + b0[...], g0[...], beta0[...]))
    h1_out[...] = _mm(x, wh[...]) + bh[...]
    xn1_out[...] = _bn(_mm(x, wn[...]) + bn_[...], gnode[...], bnode[...])


def _mid_a_body(xn_ref, part_ref, spart_ref, cpart_ref,
                we, be, gedge, bedge, gnb, bnb, w1, b1, g1, bb1,
                we_n, be_n, gedge_n, bedge_n,
                z2_out, e2_out):
    s = part_ref[0, :N, :] + part_ref[1, :N, :]
    aggr = _bn(s, gnb[...], bnb[...])
    se = spart_ref[0, :N, :] + spart_ref[1, :N, :]
    cnt = cpart_ref[0, :N, :1] + cpart_ref[1, :N, :1]
    e = _bn(_mm(se, we[...]) + cnt * be[...], gedge[...], bedge[...])
    z = jax.nn.relu(xn_ref[...] + aggr + e)
    z2_out[...] = jax.nn.relu(_bn(_mm(z, w1[...]) + b1[...], g1[...], bb1[...]))
    e2_out[...] = _bn(_mm(se, we_n[...]) + cnt * be_n[...], gedge_n[...], bedge_n[...])


def _mid_b_body(z2_ref, w2, b2, g2, bb2, wn, bn_, gnode, bnode, wh, bh,
                h2_out, xn2_out):
    x = jax.nn.relu(_bn(_mm(z2_ref[...], w2[...]) + b2[...], g2[...], bb2[...]))
    h2_out[...] = _mm(x, wh[...]) + bh[...]
    xn2_out[...] = _bn(_mm(x, wn[...]) + bn_[...], gnode[...], bnode[...])


def _dense_final_body(xn_ref, e_ref, part_ref,
                      gnb, bnb, w1, b1, g1, bb1, w2, b2, g2, bb2,
                      x_out):
    s = part_ref[0, :N, :] + part_ref[1, :N, :]
    aggr = _bn(s, gnb[...], bnb[...])
    z = jax.nn.relu(xn_ref[...] + aggr + e_ref[...])
    z = jax.nn.relu(_bn(_mm(z, w1[...]) + b1[...], g1[...], bb1[...]))
    x_out[...] = jax.nn.relu(_bn(_mm(z, w2[...]) + b2[...], g2[...], bb2[...]))


def _row(v):
    return v.reshape(1, -1)


_TC_PARAMS = pltpu.CompilerParams(vmem_limit_bytes=63 * 1024 * 1024)


def kernel(node_attr, edge_index, edge_attr, params):
    row = edge_index[0]
    col = edge_index[1]
    pad = E_PAD - E
    row_pad_tgt = N + (jnp.arange(pad, dtype=jnp.int32) % (N_PAD - N))
    col_pad_tgt = jnp.arange(pad, dtype=jnp.int32) % N
    row_p = jnp.concatenate([row, row_pad_tgt]).reshape(E_PAD // CHUNK, CHUNK)
    col_p = jnp.concatenate([col, col_pad_tgt]).reshape(E_PAD // CHUNK, CHUNK)
    zeros128 = jnp.zeros((N_PAD, EMBED), jnp.float32)
    zeros16 = jnp.zeros((N_PAD, EDGE_DIM), jnp.float32)
    ones16 = jnp.ones((CHUNK, EDGE_DIM), jnp.float32)

    p0 = params["mlp0"]
    L1, L2 = params["layers"][0], params["layers"][1]

    h1, xn1 = pl.pallas_call(
        _dense1_body,
        out_shape=(jax.ShapeDtypeStruct((N, EMBED), jnp.float32),
                   jax.ShapeDtypeStruct((N, EMBED), jnp.float32)),
        compiler_params=_TC_PARAMS,
    )(node_attr, p0["W"], _row(p0["b"]), _row(p0["g"]), _row(p0["beta"]),
      L1["Wn"], _row(L1["bn"]), _row(L1["g_node"]), _row(L1["b_node"]),
      L1["Wh"], _row(L1["bh"]))

    sparts, cparts = _edge_sum(edge_attr, row_p, ones16, zeros16)
    parts1 = _spmm(h1, col_p, row_p, zeros128)

    z2, e2 = pl.pallas_call(
        _mid_a_body,
        out_shape=(jax.ShapeDtypeStruct((N, EMBED), jnp.float32),
                   jax.ShapeDtypeStruct((N, EMBED), jnp.float32)),
        compiler_params=_TC_PARAMS,
    )(xn1, parts1, sparts, cparts,
      L1["We"], _row(L1["be"]), _row(L1["g_edge"]), _row(L1["b_edge"]),
      _row(L1["g_nb"]), _row(L1["b_nb"]),
      L1["W1"], _row(L1["b1"]), _row(L1["g1"]), _row(L1["bb1"]),
      L2["We"], _row(L2["be"]), _row(L2["g_edge"]), _row(L2["b_edge"]))

    h2, xn2 = pl.pallas_call(
        _mid_b_body,
        out_shape=(jax.ShapeDtypeStruct((N, EMBED), jnp.float32),
                   jax.ShapeDtypeStruct((N, EMBED), jnp.float32)),
        compiler_params=_TC_PARAMS,
    )(z2, L1["W2"], _row(L1["b2"]), _row(L1["g2"]), _row(L1["bb2"]),
      L2["Wn"], _row(L2["bn"]), _row(L2["g_node"]), _row(L2["b_node"]),
      L2["Wh"], _row(L2["bh"]))

    parts2 = _spmm(h2, col_p, row_p, zeros128)

    x_out = pl.pallas_call(
        _dense_final_body,
        out_shape=jax.ShapeDtypeStruct((N, EMBED), jnp.float32),
        compiler_params=_TC_PARAMS,
    )(xn2, e2, parts2,
      _row(L2["g_nb"]), _row(L2["b_nb"]),
      L2["W1"], _row(L2["b1"]), _row(L2["g1"]), _row(L2["bb1"]),
      L2["W2"], _row(L2["b2"]), _row(L2["g2"]), _row(L2["bb2"]))

    return x_out
